# baseline (speedup 1.0000x reference)
import functools

import jax
import jax.numpy as jnp
from jax import lax
from jax.experimental import pallas as pl
from jax.experimental.pallas import tpu as pltpu
from jax.experimental.pallas import tpu_sc as plsc

N = 10000
E = 320000
D = 128
H = 128
NP = 10240
NW = 32
CH = 80
EW = CH * 128
EP = NW * EW
ROWS_PER_TILE = NP // 16
K = 128
CH2 = EW // K
NBUF = 3

_mesh = plsc.VectorSubcoreMesh(core_axis_name="c", subcore_axis_name="s")


def _mm_body(x_ref, w_ref, aw2_ref, b_ref, h0_ref, s12_ref):
    h0 = jnp.dot(x_ref[...], w_ref[...], preferred_element_type=jnp.float32,
                 precision=lax.Precision.HIGHEST)
    h0_ref[...] = h0
    s12 = lax.dot_general(aw2_ref[...], h0, (((1,), (1,)), ((), ())),
                          preferred_element_type=jnp.float32,
                          precision=lax.Precision.HIGHEST)
    bias = jnp.where(lax.broadcasted_iota(jnp.int32, (2, 1), 0) == 0,
                     b_ref[0, 0], 0.0)
    s12_ref[...] = s12 + bias


def _mm_call(x_p, W_fc, aw2, b):
    return pl.pallas_call(
        _mm_body,
        out_shape=[
            jax.ShapeDtypeStruct((NP, H), jnp.float32),
            jax.ShapeDtypeStruct((2, NP), jnp.float32),
        ],
    )(x_p, W_fc, aw2, b)


@functools.partial(
    pl.kernel,
    mesh=_mesh,
    compiler_params=pltpu.CompilerParams(needs_layout_passes=False),
    out_type=[
        jax.ShapeDtypeStruct((NW, CH, 4, 128), jnp.int32),
        jax.ShapeDtypeStruct((NW, NP), jnp.float32),
    ],
    scratch_types=[
        pltpu.VMEM((CH, 128), jnp.int32),
        pltpu.VMEM((CH, 128), jnp.int32),
        pltpu.VMEM((CH, 128), jnp.int32),
        pltpu.VMEM((NP,), jnp.float32),
        pltpu.VMEM((NP,), jnp.float32),
        pltpu.VMEM((NP,), jnp.float32),
        pltpu.VMEM((CH, 4, 128), jnp.int32),
    ],
)
def _edge_score_kernel(src_hbm, dst_hbm, adj_hbm, s12_hbm, combo_hbm, parts_hbm,
                       src_v, dst_v, adj_v, s1_v, s2_v, hsum_v, combo_v):
    c = lax.axis_index("c")
    s = lax.axis_index("s")
    w = s * jnp.int32(2) + c
    pltpu.sync_copy(src_hbm.at[w], src_v)
    pltpu.sync_copy(dst_hbm.at[w], dst_v)
    pltpu.sync_copy(adj_hbm.at[w], adj_v)
    pltpu.sync_copy(s12_hbm.at[jnp.int32(0)], s1_v)
    pltpu.sync_copy(s12_hbm.at[jnp.int32(1)], s2_v)

    def zbody(i, carry):
        hsum_v[pl.ds(i * jnp.int32(16), 16)] = jnp.zeros((16,), jnp.float32)
        return carry
    lax.fori_loop(jnp.int32(0), jnp.int32(NP // 16), zbody, jnp.int32(0))

    def body(j, carry):
        for k in range(8):
            sl = pl.ds(k * 16, 16)
            src16 = src_v[j, sl]
            dst16 = dst_v[j, sl]
            z = (plsc.load_gather(s1_v, [src16])
                 + plsc.load_gather(s2_v, [dst16]))
            e16 = jnp.exp(jnp.maximum(z, 0.05 * z))
            plsc.addupdate_scatter(hsum_v, [src16], e16)
            combo_v[j, 0, sl] = src16
            combo_v[j, 1, sl] = dst16
            combo_v[j, 2, sl] = plsc.bitcast(e16, jnp.int32)
            combo_v[j, 3, sl] = adj_v[j, sl]
        return carry
    lax.fori_loop(jnp.int32(0), jnp.int32(CH), body, jnp.int32(0))

    pltpu.sync_copy(combo_v, combo_hbm.at[w])
    pltpu.sync_copy(hsum_v, parts_hbm.at[w])


@functools.partial(
    pl.kernel,
    mesh=_mesh,
    compiler_params=pltpu.CompilerParams(needs_layout_passes=False),
    out_type=[
        jax.ShapeDtypeStruct((NW, CH2, K), jnp.float32),
        jax.ShapeDtypeStruct((2, NP, H), jnp.float32),
    ],
    scratch_types=[
        pltpu.VMEM((NP,), jnp.float32),
        pltpu.VMEM((2, K, H), jnp.float32),
        pltpu.VMEM((2, 4, K), jnp.int32),
        pltpu.VMEM((K,), jnp.float32),
        pltpu.VMEM((K,), jnp.float32),
        pltpu.VMEM((16, 128), jnp.float32),
        pltpu.VMEM((128,), jnp.float32),
        pltpu.VMEM_SHARED((NP, H), jnp.float32),
        pltpu.VMEM_SHARED((NP,), jnp.float32),
        pltpu.SemaphoreType.DMA,
        pltpu.SemaphoreType.DMA,
        pltpu.SemaphoreType.DMA,
        pltpu.SemaphoreType.DMA,
        pltpu.SemaphoreType.DMA,
        pltpu.SemaphoreType.DMA,
    ],
)
def _aggregate_kernel(combo_hbm, parts_hbm, h0_hbm, alpha_hbm, outacc_hbm,
                      rec_v, rows_v, meta_v, al_v, sc_v, pv, rs, acc, rec_sh,
                      in0, in1, g0, g1, s0, s1):
    c = lax.axis_index("c")
    s = lax.axis_index("s")
    w = s * jnp.int32(2) + c
    i0 = jnp.int32(0)
    i1 = jnp.int32(1)

    def rchunk(i, carry):
        base = s * jnp.int32(ROWS_PER_TILE) + i * jnp.int32(128)
        pltpu.sync_copy(parts_hbm.at[pl.ds(jnp.int32(0), 16), pl.ds(base, 128)],
                        pv)
        for k in range(8):
            sl = pl.ds(k * 16, 16)
            su = jnp.zeros((16,), jnp.float32)
            for r in range(16):
                su = su + pv[r, sl]
            rs[sl] = su
        pltpu.sync_copy(parts_hbm.at[pl.ds(jnp.int32(16), 16), pl.ds(base, 128)],
                        pv)
        for k in range(8):
            sl = pl.ds(k * 16, 16)
            su = rs[sl]
            for r in range(16):
                su = su + pv[r, sl]
            rs[sl] = jnp.float32(1.0) / jnp.maximum(su, jnp.float32(1e-30))
        pltpu.sync_copy(rs, rec_sh.at[pl.ds(base, 128)])
        return carry
    lax.fori_loop(jnp.int32(0), jnp.int32(ROWS_PER_TILE // 128), rchunk,
                  jnp.int32(0))

    def zb(i, carry):
        for k in range(H // 16):
            rows_v[i0, i, pl.ds(k * 16, 16)] = jnp.zeros((16,), jnp.float32)
        return carry
    lax.fori_loop(jnp.int32(0), jnp.int32(K), zb, jnp.int32(0))
    for i in range(ROWS_PER_TILE // K):
        pltpu.sync_copy(rows_v.at[i0],
                        acc.at[pl.ds(s * jnp.int32(ROWS_PER_TILE) + jnp.int32(i * K), K)])
    plsc.subcore_barrier()
    pltpu.sync_copy(rec_sh, rec_v)

    def chunk_compute(j, b, bi):
        for k in range(K // 16):
            sl = pl.ds(k * 16, 16)
            src16 = meta_v[bi, 0, sl]
            r16 = plsc.load_gather(rec_v, [src16])
            e16 = plsc.bitcast(meta_v[bi, 2, sl], jnp.float32)
            adj16 = plsc.bitcast(meta_v[bi, 3, sl], jnp.float32)
            a16 = e16 * r16
            al_v[sl] = a16
            sc_v[sl] = a16 * adj16
        pltpu.sync_copy(al_v, alpha_hbm.at[w, j])

        def rbody(t, rcarry):
            base = t * jnp.int32(16)
            sc16 = sc_v[pl.ds(base, 16)]
            for q in range(16):
                aq = jnp.full((16,), sc16[q], jnp.float32)
                r = base + jnp.int32(q)
                for k in range(H // 16):
                    rows_v[b, r, pl.ds(k * 16, 16)] = (
                        rows_v[b, r, pl.ds(k * 16, 16)] * aq)
            return rcarry
        lax.fori_loop(jnp.int32(0), jnp.int32(K // 16), rbody, jnp.int32(0))

    def cbody(g, carry):
        j0 = g * jnp.int32(2)
        j1 = j0 + i1
        cin0 = pltpu.async_copy(combo_hbm.at[w, j0], meta_v.at[i0], in0)
        cin1 = pltpu.async_copy(combo_hbm.at[w, j1], meta_v.at[i1], in1)
        cin0.wait()
        cg0 = pltpu.async_copy(h0_hbm.at[meta_v.at[i0, i1]], rows_v.at[i0], g0)
        cin1.wait()
        cg1 = pltpu.async_copy(h0_hbm.at[meta_v.at[i1, i1]], rows_v.at[i1], g1)
        cg0.wait()
        chunk_compute(j0, i0, 0)
        cs0 = pltpu.async_copy(rows_v.at[i0], acc.at[meta_v.at[i0, i0]], s0,
                               add=True)
        cg1.wait()
        chunk_compute(j1, i1, 1)
        cs1 = pltpu.async_copy(rows_v.at[i1], acc.at[meta_v.at[i1, i0]], s1,
                               add=True)
        cs0.wait()
        cs1.wait()
        return carry
    lax.fori_loop(jnp.int32(0), jnp.int32(CH2 // 2), cbody, jnp.int32(0))

    plsc.subcore_barrier()
    for i in range(ROWS_PER_TILE // 128):
        base = s * jnp.int32(ROWS_PER_TILE) + jnp.int32(i * 128)
        pltpu.sync_copy(acc.at[pl.ds(base, 128)],
                        outacc_hbm.at[c, pl.ds(base, 128)])


def _add_body(acc_ref, out_ref):
    a = acc_ref[...]
    out_ref[...] = a[0, :N, :] + a[1, :N, :]


def _add_call(outacc):
    return pl.pallas_call(
        _add_body,
        out_shape=jax.ShapeDtypeStruct((N, H), jnp.float32),
    )(outacc)


def kernel(x, edge_index, adj_values, W_fc, a_w, a_b):
    src = edge_index[0].astype(jnp.int32)
    dst = edge_index[1].astype(jnp.int32)
    pad = N + (jnp.arange(EP - E, dtype=jnp.int32) % (NP - N))
    src_p = jnp.concatenate([src, pad]).reshape(NW, CH, 128)
    dst_p = jnp.concatenate([dst, pad]).reshape(NW, CH, 128)
    adj_p = jnp.concatenate(
        [adj_values.astype(jnp.float32), jnp.zeros((EP - E,), jnp.float32)]
    ).reshape(NW, CH, 128)
    x_p = jnp.pad(x.astype(jnp.float32), ((0, NP - N), (0, 0)))
    aw2 = a_w.astype(jnp.float32).reshape(2, H)
    b = a_b.astype(jnp.float32).reshape(1, 1)

    adj32 = lax.bitcast_convert_type(adj_p, jnp.int32)
    h0_p, s12 = _mm_call(x_p, W_fc.astype(jnp.float32), aw2, b)
    combo, parts = _edge_score_kernel(src_p, dst_p, adj32, s12)
    alpha_p, outacc = _aggregate_kernel(combo, parts, h0_p)
    out = _add_call(outacc)
    alpha = alpha_p.reshape(-1)[:E]
    return (out.astype(jnp.float64), alpha.astype(jnp.float64))


# speedup vs baseline: 1.0285x; 1.0285x over previous
import functools

import jax
import jax.numpy as jnp
from jax import lax
from jax.experimental import pallas as pl
from jax.experimental.pallas import tpu as pltpu
from jax.experimental.pallas import tpu_sc as plsc

N = 10000
E = 320000
D = 128
H = 128
NP = 10240
NW = 32
CH = 80
EW = CH * 128
EP = NW * EW
ROWS_PER_TILE = NP // 16
K = 128
CH2 = EW // K
NBUF = 3

_mesh = plsc.VectorSubcoreMesh(core_axis_name="c", subcore_axis_name="s")


def _score_body(x_ref, w_ref, aw2_ref, b_ref, s12_ref):
    ws2 = lax.dot_general(aw2_ref[...], w_ref[...], (((1,), (1,)), ((), ())),
                          preferred_element_type=jnp.float32,
                          precision=lax.Precision.HIGHEST)
    s12 = lax.dot_general(ws2, x_ref[...], (((1,), (1,)), ((), ())),
                          preferred_element_type=jnp.float32,
                          precision=lax.Precision.HIGHEST)
    bias = jnp.where(lax.broadcasted_iota(jnp.int32, (2, 1), 0) == 0,
                     b_ref[0, 0], 0.0)
    s12_ref[...] = s12 + bias


def _score_call(x_p, W_fc, aw2, b):
    return pl.pallas_call(
        _score_body,
        out_shape=jax.ShapeDtypeStruct((2, NP), jnp.float32),
    )(x_p, W_fc, aw2, b)


def _h0_body(x_ref, w_ref, h0_ref):
    h0_ref[...] = jnp.dot(x_ref[...], w_ref[...],
                          preferred_element_type=jnp.float32,
                          precision=lax.Precision.HIGHEST)


def _h0_call(x_p, W_fc):
    return pl.pallas_call(
        _h0_body,
        out_shape=jax.ShapeDtypeStruct((NP, H), jnp.float32),
    )(x_p, W_fc)


@functools.partial(
    pl.kernel,
    mesh=_mesh,
    compiler_params=pltpu.CompilerParams(needs_layout_passes=False),
    out_type=[
        jax.ShapeDtypeStruct((NW, CH, 4, 128), jnp.int32),
        jax.ShapeDtypeStruct((NW, NP), jnp.float32),
    ],
    scratch_types=[
        pltpu.VMEM((CH, 128), jnp.int32),
        pltpu.VMEM((CH, 128), jnp.int32),
        pltpu.VMEM((CH, 128), jnp.int32),
        pltpu.VMEM((NP,), jnp.float32),
        pltpu.VMEM((NP,), jnp.float32),
        pltpu.VMEM((NP,), jnp.float32),
        pltpu.VMEM((CH, 4, 128), jnp.int32),
    ],
)
def _edge_score_kernel(src_hbm, dst_hbm, adj_hbm, s12_hbm, combo_hbm, parts_hbm,
                       src_v, dst_v, adj_v, s1_v, s2_v, hsum_v, combo_v):
    c = lax.axis_index("c")
    s = lax.axis_index("s")
    w = s * jnp.int32(2) + c
    pltpu.sync_copy(src_hbm.at[w], src_v)
    pltpu.sync_copy(dst_hbm.at[w], dst_v)
    pltpu.sync_copy(adj_hbm.at[w], adj_v)
    pltpu.sync_copy(s12_hbm.at[jnp.int32(0)], s1_v)
    pltpu.sync_copy(s12_hbm.at[jnp.int32(1)], s2_v)

    def zbody(i, carry):
        hsum_v[pl.ds(i * jnp.int32(16), 16)] = jnp.zeros((16,), jnp.float32)
        return carry
    lax.fori_loop(jnp.int32(0), jnp.int32(NP // 16), zbody, jnp.int32(0))

    def body(j, carry):
        for k in range(8):
            sl = pl.ds(k * 16, 16)
            src16 = src_v[j, sl]
            dst16 = dst_v[j, sl]
            z = (plsc.load_gather(s1_v, [src16])
                 + plsc.load_gather(s2_v, [dst16]))
            e16 = jnp.exp(jnp.maximum(z, 0.05 * z))
            plsc.addupdate_scatter(hsum_v, [src16], e16)
            combo_v[j, 0, sl] = src16
            combo_v[j, 1, sl] = dst16
            combo_v[j, 2, sl] = plsc.bitcast(e16, jnp.int32)
            combo_v[j, 3, sl] = adj_v[j, sl]
        return carry
    lax.fori_loop(jnp.int32(0), jnp.int32(CH), body, jnp.int32(0))

    pltpu.sync_copy(combo_v, combo_hbm.at[w])
    pltpu.sync_copy(hsum_v, parts_hbm.at[w])


@functools.partial(
    pl.kernel,
    mesh=_mesh,
    compiler_params=pltpu.CompilerParams(needs_layout_passes=False),
    out_type=[
        jax.ShapeDtypeStruct((NW, CH2, K), jnp.float32),
        jax.ShapeDtypeStruct((2, NP, H), jnp.float32),
    ],
    scratch_types=[
        pltpu.VMEM((NP,), jnp.float32),
        pltpu.VMEM((2, K, H), jnp.float32),
        pltpu.VMEM((2, 4, K), jnp.int32),
        pltpu.VMEM((K,), jnp.float32),
        pltpu.VMEM((K,), jnp.float32),
        pltpu.VMEM((16, 128), jnp.float32),
        pltpu.VMEM((128,), jnp.float32),
        pltpu.VMEM_SHARED((NP, H), jnp.float32),
        pltpu.VMEM_SHARED((NP,), jnp.float32),
        pltpu.SemaphoreType.DMA,
        pltpu.SemaphoreType.DMA,
        pltpu.SemaphoreType.DMA,
        pltpu.SemaphoreType.DMA,
        pltpu.SemaphoreType.DMA,
        pltpu.SemaphoreType.DMA,
    ],
)
def _aggregate_kernel(combo_hbm, parts_hbm, h0_hbm, alpha_hbm, outacc_hbm,
                      rec_v, rows_v, meta_v, al_v, sc_v, pv, rs, acc, rec_sh,
                      in0, in1, g0, g1, s0, s1):
    c = lax.axis_index("c")
    s = lax.axis_index("s")
    w = s * jnp.int32(2) + c
    i0 = jnp.int32(0)
    i1 = jnp.int32(1)

    def rchunk(i, carry):
        base = s * jnp.int32(ROWS_PER_TILE) + i * jnp.int32(128)
        pltpu.sync_copy(parts_hbm.at[pl.ds(jnp.int32(0), 16), pl.ds(base, 128)],
                        pv)
        for k in range(8):
            sl = pl.ds(k * 16, 16)
            su = jnp.zeros((16,), jnp.float32)
            for r in range(16):
                su = su + pv[r, sl]
            rs[sl] = su
        pltpu.sync_copy(parts_hbm.at[pl.ds(jnp.int32(16), 16), pl.ds(base, 128)],
                        pv)
        for k in range(8):
            sl = pl.ds(k * 16, 16)
            su = rs[sl]
            for r in range(16):
                su = su + pv[r, sl]
            rs[sl] = jnp.float32(1.0) / jnp.maximum(su, jnp.float32(1e-30))
        pltpu.sync_copy(rs, rec_sh.at[pl.ds(base, 128)])
        return carry
    lax.fori_loop(jnp.int32(0), jnp.int32(ROWS_PER_TILE // 128), rchunk,
                  jnp.int32(0))

    def zb(i, carry):
        for k in range(H // 16):
            rows_v[i0, i, pl.ds(k * 16, 16)] = jnp.zeros((16,), jnp.float32)
        return carry
    lax.fori_loop(jnp.int32(0), jnp.int32(K), zb, jnp.int32(0))
    for i in range(ROWS_PER_TILE // K):
        pltpu.sync_copy(rows_v.at[i0],
                        acc.at[pl.ds(s * jnp.int32(ROWS_PER_TILE) + jnp.int32(i * K), K)])
    plsc.subcore_barrier()
    pltpu.sync_copy(rec_sh, rec_v)

    def chunk_compute(j, b, bi):
        for k in range(K // 16):
            sl = pl.ds(k * 16, 16)
            src16 = meta_v[bi, 0, sl]
            r16 = plsc.load_gather(rec_v, [src16])
            e16 = plsc.bitcast(meta_v[bi, 2, sl], jnp.float32)
            adj16 = plsc.bitcast(meta_v[bi, 3, sl], jnp.float32)
            a16 = e16 * r16
            al_v[sl] = a16
            sc_v[sl] = a16 * adj16
        pltpu.sync_copy(al_v, alpha_hbm.at[w, j])

        def rbody(t, rcarry):
            base = t * jnp.int32(16)
            sc16 = sc_v[pl.ds(base, 16)]
            for q in range(16):
                aq = jnp.full((16,), sc16[q], jnp.float32)
                r = base + jnp.int32(q)
                for k in range(H // 16):
                    rows_v[b, r, pl.ds(k * 16, 16)] = (
                        rows_v[b, r, pl.ds(k * 16, 16)] * aq)
            return rcarry
        lax.fori_loop(jnp.int32(0), jnp.int32(K // 16), rbody, jnp.int32(0))

    def cbody(g, carry):
        j0 = g * jnp.int32(2)
        j1 = j0 + i1
        cin0 = pltpu.async_copy(combo_hbm.at[w, j0], meta_v.at[i0], in0)
        cin1 = pltpu.async_copy(combo_hbm.at[w, j1], meta_v.at[i1], in1)
        cin0.wait()
        cg0 = pltpu.async_copy(h0_hbm.at[meta_v.at[i0, i1]], rows_v.at[i0], g0)
        cin1.wait()
        cg1 = pltpu.async_copy(h0_hbm.at[meta_v.at[i1, i1]], rows_v.at[i1], g1)
        cg0.wait()
        chunk_compute(j0, i0, 0)
        cs0 = pltpu.async_copy(rows_v.at[i0], acc.at[meta_v.at[i0, i0]], s0,
                               add=True)
        cg1.wait()
        chunk_compute(j1, i1, 1)
        cs1 = pltpu.async_copy(rows_v.at[i1], acc.at[meta_v.at[i1, i0]], s1,
                               add=True)
        cs0.wait()
        cs1.wait()
        return carry
    lax.fori_loop(jnp.int32(0), jnp.int32(CH2 // 2), cbody, jnp.int32(0))

    plsc.subcore_barrier()
    for i in range(ROWS_PER_TILE // 128):
        base = s * jnp.int32(ROWS_PER_TILE) + jnp.int32(i * 128)
        pltpu.sync_copy(acc.at[pl.ds(base, 128)],
                        outacc_hbm.at[c, pl.ds(base, 128)])


def _add_body(acc_ref, out_ref):
    a = acc_ref[...]
    out_ref[...] = a[0, :N, :] + a[1, :N, :]


def _add_call(outacc):
    return pl.pallas_call(
        _add_body,
        out_shape=jax.ShapeDtypeStruct((N, H), jnp.float32),
    )(outacc)


def kernel(x, edge_index, adj_values, W_fc, a_w, a_b):
    src = edge_index[0].astype(jnp.int32)
    dst = edge_index[1].astype(jnp.int32)
    pad = N + (jnp.arange(EP - E, dtype=jnp.int32) % (NP - N))
    src_p = jnp.concatenate([src, pad]).reshape(NW, CH, 128)
    dst_p = jnp.concatenate([dst, pad]).reshape(NW, CH, 128)
    adj_p = jnp.concatenate(
        [adj_values.astype(jnp.float32), jnp.zeros((EP - E,), jnp.float32)]
    ).reshape(NW, CH, 128)
    x_p = jnp.pad(x.astype(jnp.float32), ((0, NP - N), (0, 0)))
    aw2 = a_w.astype(jnp.float32).reshape(2, H)
    b = a_b.astype(jnp.float32).reshape(1, 1)

    adj32 = lax.bitcast_convert_type(adj_p, jnp.int32)
    W32 = W_fc.astype(jnp.float32)
    s12 = _score_call(x_p, W32, aw2, b)
    combo, parts = _edge_score_kernel(src_p, dst_p, adj32, s12)
    h0_p = _h0_call(x_p, W32)
    alpha_p, outacc = _aggregate_kernel(combo, parts, h0_p)
    out = _add_call(outacc)
    alpha = alpha_p.reshape(-1)[:E]
    return (out.astype(jnp.float64), alpha.astype(jnp.float64))
